# Initial kernel scaffold; baseline (speedup 1.0000x reference)
import jax
import jax.numpy as jnp
from jax.experimental import pallas as pl


def kernel(x, tables):
    raise NotImplementedError("write your pallas kernel here")


# speedup vs baseline: 1.1421x; 1.1421x over previous
import functools

import jax
import jax.numpy as jnp
from jax import lax
from jax.experimental import pallas as pl
from jax.experimental.pallas import tpu as pltpu
from jax.experimental.pallas import tpu_sc as plsc

N_CAT = 26
VOCAB = 100000
EMB = 16
BATCH = 16384
NUM = 13
OUT_DIM = N_CAT * EMB + NUM

NC, NS = 2, 16
NW = NC * NS
ROWS_W = BATCH // NW
SUB = 128
NSUB = ROWS_W // SUB


@functools.partial(
    pl.kernel,
    out_type=jax.ShapeDtypeStruct((BATCH, OUT_DIM), jnp.float32),
    mesh=plsc.VectorSubcoreMesh(core_axis_name="c", subcore_axis_name="s"),
    compiler_params=pltpu.CompilerParams(use_tc_tiling_on_sc=False),
    scratch_types=[
        pltpu.VMEM((N_CAT, SUB), jnp.int32),
        pltpu.VMEM((N_CAT, SUB, EMB), jnp.float32),
        pltpu.VMEM((SUB, NUM), jnp.float32),
        pltpu.SemaphoreType.DMA,
    ],
)
def _emb_kernel(table_hbm, idx_hbm, num_hbm, out_hbm, idx_v, emb_v, num_v, sem):
    wid = lax.axis_index("s") * NC + lax.axis_index("c")

    def body(j):
        base = wid * ROWS_W + j * SUB
        pltpu.sync_copy(idx_hbm.at[:, pl.ds(base, SUB)], idx_v)
        gathers = [
            pltpu.async_copy(table_hbm.at[idx_v.at[i]], emb_v.at[i], sem)
            for i in range(N_CAT)
        ]
        pltpu.sync_copy(num_hbm.at[pl.ds(base, SUB)], num_v)
        pltpu.sync_copy(
            num_v, out_hbm.at[pl.ds(base, SUB), pl.ds(N_CAT * EMB, NUM)]
        )
        for c in gathers:
            c.wait()
        stores = [
            pltpu.async_copy(
                emb_v.at[i],
                out_hbm.at[pl.ds(base, SUB), pl.ds(i * EMB, EMB)],
                sem,
            )
            for i in range(N_CAT)
        ]
        for c in stores:
            c.wait()

    pl.loop(0, NSUB)(body)


def kernel(x, tables):
    cat = x[:, :N_CAT].astype(jnp.int32)
    idx = cat.T + jnp.arange(N_CAT, dtype=jnp.int32)[:, None] * VOCAB
    num = x[:, N_CAT:]
    flat_tables = tables.reshape(N_CAT * VOCAB, EMB)
    return _emb_kernel(flat_tables, idx, num)
